# baseline (speedup 1.0000x reference)
import functools

import jax
import jax.numpy as jnp
from jax import lax
from jax.experimental import pallas as pl
from jax.experimental.pallas import tpu as pltpu
from jax.experimental.pallas import tpu_sc as plsc

B, C, H, W = 4, 19, 512, 512
N = B * H * W
K = 2048
TRASH = 2 * K
S = 4225
HWORDS = 16 * S
SEG_PER_CLASS = 8
SEG = N // SEG_PER_CLASS
NSEG = C * SEG_PER_CLASS
NWORKERS = 32
ROUNDS = (NSEG + NWORKERS - 1) // NWORKERS
CH = 16384
NPAIR = SEG // (2 * CH)
ROWS_BLK = 64


def _encode_body(logits_ref, labels_ref, out_ref):
    x = logits_ref[0]
    lab = labels_ref[0]
    m = jnp.max(x, axis=0)
    ex = jnp.exp(x - m[None])
    inv = 1.0 / jnp.sum(ex, axis=0)
    p = ex * inv[None]
    valid = lab != 0
    cidx = lax.broadcasted_iota(jnp.int32, (C, ROWS_BLK, W), 0)
    fg = (lab[None] == cidx) & valid[None]
    e = jnp.where(fg, 1.0 - p, p)
    bucket = jnp.clip((e * float(K)).astype(jnp.int32), 0, K - 1)
    code = jnp.where(fg, K + bucket, bucket)
    code = jnp.where(valid[None], code, TRASH)
    wio = lax.broadcasted_iota(jnp.int32, (ROWS_BLK, W), 1)
    lane = (wio % 16 + (wio // 16)) % 16
    out_ref[:, 0] = code + (lane * S)[None]


def _encode(logits, labels):
    grid = (B, H // ROWS_BLK)
    return pl.pallas_call(
        _encode_body,
        grid=grid,
        in_specs=[
            pl.BlockSpec((1, C, ROWS_BLK, W), lambda b, r: (b, 0, r, 0)),
            pl.BlockSpec((1, ROWS_BLK, W), lambda b, r: (b, r, 0)),
        ],
        out_specs=pl.BlockSpec((C, 1, ROWS_BLK, W), lambda b, r: (0, b, r, 0)),
        out_shape=jax.ShapeDtypeStruct((C, B, H, W), jnp.int32),
    )(logits, labels)


def _sc_hist_body(sidx_hbm, out_hbm, hist, buf0, buf1, sem0, sem1):
    wid = lax.axis_index("s") * 2 + lax.axis_index("c")
    ones = jnp.ones((16,), jnp.int32)
    zeros = jnp.zeros((16,), jnp.int32)

    def zero8(i, _):
        for u in range(8):
            hist[pl.ds(i * 128 + u * 16, 16)] = zeros
        return _

    def zero_hist():
        lax.fori_loop(0, HWORDS // 128, zero8, None)
        for u in range(HWORDS // 128 * 128, HWORDS, 16):
            hist[pl.ds(u, 16)] = zeros

    def scat_all(buf):
        def scat8(i, _):
            b0 = i * 128
            iot = lax.iota(jnp.int32, 16) * S
            for u in range(8):
                idx = buf[pl.ds(b0 + u * 16, 16)]
                idx = iot + (idx >> 31) + u
                plsc.addupdate_scatter(hist, [idx], ones)
            return _

        lax.fori_loop(0, CH // 128, scat8, None)

    def wait_dma(buf, sem):
        pltpu.make_async_copy(sidx_hbm.at[pl.ds(0, CH)], buf, sem).wait()

    zero_hist()

    for r in range(ROUNDS):
        s = wid + r * NWORKERS

        @pl.when(s < NSEG)
        def _():
            base = s * SEG
            pltpu.async_copy(sidx_hbm.at[pl.ds(base, CH)], buf0, sem0)

            def pair(jp, _):
                off = base + jp * (2 * CH)
                pltpu.async_copy(sidx_hbm.at[pl.ds(off + CH, CH)], buf1, sem1)
                wait_dma(buf0, sem0)
                scat_all(buf0)

                @pl.when(jp < NPAIR - 1)
                def _():
                    pltpu.async_copy(
                        sidx_hbm.at[pl.ds(off + 2 * CH, CH)], buf0, sem0)

                wait_dma(buf1, sem1)
                scat_all(buf1)
                return _

            lax.fori_loop(0, NPAIR, pair, None)
            pltpu.sync_copy(hist, out_hbm.at[s])
            zero_hist()


def _sc_hist(sidx_flat):
    mesh = plsc.VectorSubcoreMesh(core_axis_name="c", subcore_axis_name="s")
    fn = functools.partial(
        pl.kernel,
        mesh=mesh,
        out_type=jax.ShapeDtypeStruct((NSEG, HWORDS), jnp.int32),
        scratch_types=[
            pltpu.VMEM((HWORDS,), jnp.int32),
            pltpu.VMEM((CH,), jnp.int32),
            pltpu.VMEM((CH,), jnp.int32),
            pltpu.SemaphoreType.DMA,
            pltpu.SemaphoreType.DMA,
        ],
        compiler_params=pltpu.CompilerParams(
            needs_layout_passes=False, use_tc_tiling_on_sc=True),
    )(_sc_hist_body)
    return fn(sidx_flat)


def _finalize_body(part_ref, out_ref, acc_ref):
    c = pl.program_id(0)
    q = part_ref[0].astype(jnp.float32)
    cnt = jnp.sum(q, axis=0)
    f2 = cnt[K:2 * K].reshape(16, 128)
    n2 = cnt[0:K].reshape(16, 128) + f2

    io = lax.broadcasted_iota(jnp.int32, (128, 128), 0)
    jo = lax.broadcasted_iota(jnp.int32, (128, 128), 1)
    tri_incl = (io >= jo).astype(jnp.float32)
    wn = jnp.dot(n2, tri_incl, preferred_element_type=jnp.float32)
    wf = jnp.dot(f2, tri_incl, preferred_element_type=jnp.float32)
    totn = wn[:, 0:1]
    totf = wf[:, 0:1]
    ia = lax.broadcasted_iota(jnp.int32, (16, 16), 0)
    ja = lax.broadcasted_iota(jnp.int32, (16, 16), 1)
    tri_after = (ja > ia).astype(jnp.float32)
    san = jnp.dot(tri_after, totn, preferred_element_type=jnp.float32)
    saf = jnp.dot(tri_after, totf, preferred_element_type=jnp.float32)
    nn = wn + san
    pp = wf + saf
    g = pp[0, 0]
    jac = 1.0 - (g - pp) / jnp.maximum(g + nn - pp, 1.0)
    loss_c = (jnp.sum(jac) - 0.5 * jac[0, 0]) * (1.0 / float(K))
    present = (g > 0.0).astype(jnp.float32)

    @pl.when(c == 0)
    def _():
        acc_ref[0] = 0.0
        acc_ref[1] = 0.0
        out_ref[0, 0] = 0.0

    acc_ref[0] += loss_c * present
    acc_ref[1] += present

    @pl.when(c == C - 1)
    def _():
        out_ref[0, 0] = acc_ref[0] / jnp.maximum(acc_ref[1], 1.0)


def _finalize(part):
    return pl.pallas_call(
        _finalize_body,
        grid=(C,),
        in_specs=[pl.BlockSpec((1, 128, S), lambda c: (c, 0, 0))],
        out_specs=pl.BlockSpec(memory_space=pltpu.SMEM),
        out_shape=jax.ShapeDtypeStruct((1, 1), jnp.float32),
        scratch_shapes=[pltpu.SMEM((2,), jnp.float32)],
    )(part)


def kernel(logits, labels):
    labels = labels.astype(jnp.int32)
    sidx = _encode(logits, labels)
    part = _sc_hist(sidx.reshape(-1))
    loss = _finalize(part.reshape(C, 128, S))
    return loss[0, 0]


# speedup vs baseline: 1.7551x; 1.7551x over previous
import functools

import jax
import jax.numpy as jnp
from jax import lax
from jax.experimental import pallas as pl
from jax.experimental.pallas import tpu as pltpu
from jax.experimental.pallas import tpu_sc as plsc

B, C, H, W = 4, 19, 512, 512
N = B * H * W
K = 2048
TRASH = 2 * K
S = 4225
HWORDS = 16 * S
SEG_PER_CLASS = 8
SEG = N // SEG_PER_CLASS
NSEG = C * SEG_PER_CLASS
NWORKERS = 32
ROUNDS = (NSEG + NWORKERS - 1) // NWORKERS
CH = 16384
NPAIR = SEG // (2 * CH)
ROWS_BLK = 64


def _encode_body(logits_ref, labels_ref, out_ref):
    x = logits_ref[0]
    lab = labels_ref[0]
    m = jnp.max(x, axis=0)
    ex = jnp.exp(x - m[None])
    inv = 1.0 / jnp.sum(ex, axis=0)
    p = ex * inv[None]
    valid = lab != 0
    cidx = lax.broadcasted_iota(jnp.int32, (C, ROWS_BLK, W), 0)
    fg = (lab[None] == cidx) & valid[None]
    e = jnp.where(fg, 1.0 - p, p)
    bucket = jnp.clip((e * float(K)).astype(jnp.int32), 0, K - 1)
    code = jnp.where(fg, K + bucket, bucket)
    code = jnp.where(valid[None], code, TRASH)
    wio = lax.broadcasted_iota(jnp.int32, (ROWS_BLK, W), 1)
    lane = (wio % 16 + (wio // 16)) % 16
    out_ref[:, 0] = code + (lane * S)[None]


def _encode(logits, labels):
    grid = (B, H // ROWS_BLK)
    return pl.pallas_call(
        _encode_body,
        grid=grid,
        in_specs=[
            pl.BlockSpec((1, C, ROWS_BLK, W), lambda b, r: (b, 0, r, 0)),
            pl.BlockSpec((1, ROWS_BLK, W), lambda b, r: (b, r, 0)),
        ],
        out_specs=pl.BlockSpec((C, 1, ROWS_BLK, W), lambda b, r: (0, b, r, 0)),
        out_shape=jax.ShapeDtypeStruct((C, B, H, W), jnp.int32),
    )(logits, labels)


def _sc_hist_body(sidx_hbm, out_hbm, hist, buf0, buf1, sem0, sem1):
    wid = lax.axis_index("s") * 2 + lax.axis_index("c")
    ones = jnp.ones((16,), jnp.int32)
    zeros = jnp.zeros((16,), jnp.int32)

    def zero8(i, _):
        for u in range(8):
            hist[pl.ds(i * 128 + u * 16, 16)] = zeros
        return _

    def zero_hist():
        lax.fori_loop(0, HWORDS // 128, zero8, None)
        for u in range(HWORDS // 128 * 128, HWORDS, 16):
            hist[pl.ds(u, 16)] = zeros

    def scat_all(buf):
        @plsc.parallel_loop(0, CH // 16, unroll=8)
        def _(i):
            idx = buf[pl.ds(i * 16, 16)]
            plsc.addupdate_scatter(hist, [idx], ones)

    def wait_dma(buf, sem):
        pltpu.make_async_copy(sidx_hbm.at[pl.ds(0, CH)], buf, sem).wait()

    zero_hist()

    for r in range(ROUNDS):
        s = wid + r * NWORKERS

        @pl.when(s < NSEG)
        def _():
            base = s * SEG
            pltpu.async_copy(sidx_hbm.at[pl.ds(base, CH)], buf0, sem0)

            def pair(jp, _):
                off = base + jp * (2 * CH)
                pltpu.async_copy(sidx_hbm.at[pl.ds(off + CH, CH)], buf1, sem1)
                wait_dma(buf0, sem0)
                scat_all(buf0)

                @pl.when(jp < NPAIR - 1)
                def _():
                    pltpu.async_copy(
                        sidx_hbm.at[pl.ds(off + 2 * CH, CH)], buf0, sem0)

                wait_dma(buf1, sem1)
                scat_all(buf1)
                return _

            lax.fori_loop(0, NPAIR, pair, None)
            pltpu.sync_copy(hist, out_hbm.at[s])
            zero_hist()


def _sc_hist(sidx_flat):
    mesh = plsc.VectorSubcoreMesh(core_axis_name="c", subcore_axis_name="s")
    fn = functools.partial(
        pl.kernel,
        mesh=mesh,
        out_type=jax.ShapeDtypeStruct((NSEG, HWORDS), jnp.int32),
        scratch_types=[
            pltpu.VMEM((HWORDS,), jnp.int32),
            pltpu.VMEM((CH,), jnp.int32),
            pltpu.VMEM((CH,), jnp.int32),
            pltpu.SemaphoreType.DMA,
            pltpu.SemaphoreType.DMA,
        ],
        compiler_params=pltpu.CompilerParams(
            needs_layout_passes=False, use_tc_tiling_on_sc=True),
    )(_sc_hist_body)
    return fn(sidx_flat)


def _finalize_body(part_ref, out_ref, acc_ref):
    c = pl.program_id(0)
    q = part_ref[0].astype(jnp.float32)
    cnt = jnp.sum(q, axis=0)
    f2 = cnt[K:2 * K].reshape(16, 128)
    n2 = cnt[0:K].reshape(16, 128) + f2

    io = lax.broadcasted_iota(jnp.int32, (128, 128), 0)
    jo = lax.broadcasted_iota(jnp.int32, (128, 128), 1)
    tri_incl = (io >= jo).astype(jnp.float32)
    wn = jnp.dot(n2, tri_incl, preferred_element_type=jnp.float32)
    wf = jnp.dot(f2, tri_incl, preferred_element_type=jnp.float32)
    totn = wn[:, 0:1]
    totf = wf[:, 0:1]
    ia = lax.broadcasted_iota(jnp.int32, (16, 16), 0)
    ja = lax.broadcasted_iota(jnp.int32, (16, 16), 1)
    tri_after = (ja > ia).astype(jnp.float32)
    san = jnp.dot(tri_after, totn, preferred_element_type=jnp.float32)
    saf = jnp.dot(tri_after, totf, preferred_element_type=jnp.float32)
    nn = wn + san
    pp = wf + saf
    g = pp[0, 0]
    jac = 1.0 - (g - pp) / jnp.maximum(g + nn - pp, 1.0)
    loss_c = (jnp.sum(jac) - 0.5 * jac[0, 0]) * (1.0 / float(K))
    present = (g > 0.0).astype(jnp.float32)

    @pl.when(c == 0)
    def _():
        acc_ref[0] = 0.0
        acc_ref[1] = 0.0
        out_ref[0, 0] = 0.0

    acc_ref[0] += loss_c * present
    acc_ref[1] += present

    @pl.when(c == C - 1)
    def _():
        out_ref[0, 0] = acc_ref[0] / jnp.maximum(acc_ref[1], 1.0)


def _finalize(part):
    return pl.pallas_call(
        _finalize_body,
        grid=(C,),
        in_specs=[pl.BlockSpec((1, 128, S), lambda c: (c, 0, 0))],
        out_specs=pl.BlockSpec(memory_space=pltpu.SMEM),
        out_shape=jax.ShapeDtypeStruct((1, 1), jnp.float32),
        scratch_shapes=[pltpu.SMEM((2,), jnp.float32)],
    )(part)


def kernel(logits, labels):
    labels = labels.astype(jnp.int32)
    sidx = _encode(logits, labels)
    part = _sc_hist(sidx.reshape(-1))
    loss = _finalize(part.reshape(C, 128, S))
    return loss[0, 0]


# speedup vs baseline: 2.1166x; 1.2060x over previous
import functools

import jax
import jax.numpy as jnp
from jax import lax
from jax.experimental import pallas as pl
from jax.experimental.pallas import tpu as pltpu
from jax.experimental.pallas import tpu_sc as plsc

B, C, H, W = 4, 19, 512, 512
N = B * H * W
K = 2048
TRASH = 2 * K
S = 4225
HWORDS = 16 * S
SEG_PER_CLASS = 8
SEG = N // SEG_PER_CLASS
NSEG = C * SEG_PER_CLASS
NWORKERS = 32
ROUNDS = (NSEG + NWORKERS - 1) // NWORKERS
CH = 16384
NPAIR = SEG // (2 * CH)
ROWS_BLK = 64


def _encode_body(logits_ref, labels_ref, out_ref):
    x = logits_ref[0]
    lab = labels_ref[0]
    m = jnp.max(x, axis=0)
    ex = jnp.exp(x - m[None])
    inv = 1.0 / jnp.sum(ex, axis=0)
    p = ex * inv[None]
    valid = lab != 0
    cidx = lax.broadcasted_iota(jnp.int32, (C, ROWS_BLK, W), 0)
    fg = (lab[None] == cidx) & valid[None]
    e = jnp.where(fg, 1.0 - p, p)
    bucket = jnp.clip((e * float(K)).astype(jnp.int32), 0, K - 1)
    code = jnp.where(fg, K + bucket, bucket)
    code = jnp.where(valid[None], code, TRASH)
    wio = lax.broadcasted_iota(jnp.int32, (ROWS_BLK, W), 1)
    lane = (wio % 16 + (wio // 16)) % 16
    sidx = code + (lane * S)[None]
    out_ref[...] = sidx.reshape(C, ROWS_BLK * W // 128, 128)


def _encode(logits, labels):
    grid = (B, H // ROWS_BLK)
    return pl.pallas_call(
        _encode_body,
        grid=grid,
        in_specs=[
            pl.BlockSpec((1, C, ROWS_BLK, W), lambda b, r: (b, 0, r, 0)),
            pl.BlockSpec((1, ROWS_BLK, W), lambda b, r: (b, r, 0)),
        ],
        out_specs=pl.BlockSpec(
            (C, ROWS_BLK * W // 128, 128),
            lambda b, r: (0, b * (H // ROWS_BLK) + r, 0)),
        out_shape=jax.ShapeDtypeStruct((C, N // 128, 128), jnp.int32),
    )(logits, labels)


def _sc_hist_body(sidx_hbm, out_hbm, hist, buf0, buf1, sem0, sem1):
    wid = lax.axis_index("s") * 2 + lax.axis_index("c")
    ones = jnp.ones((16,), jnp.int32)
    zeros = jnp.zeros((16,), jnp.int32)

    def zero8(i, _):
        for u in range(8):
            hist[pl.ds(i * 128 + u * 16, 16)] = zeros
        return _

    def zero_hist():
        lax.fori_loop(0, HWORDS // 128, zero8, None)
        for u in range(HWORDS // 128 * 128, HWORDS, 16):
            hist[pl.ds(u, 16)] = zeros

    def scat_all(buf):
        @plsc.parallel_loop(0, CH // 16, unroll=8)
        def _(i):
            idx = buf[pl.ds(i * 16, 16)]
            plsc.addupdate_scatter(hist, [idx], ones)

    def wait_dma(buf, sem):
        pltpu.make_async_copy(sidx_hbm.at[pl.ds(0, CH)], buf, sem).wait()

    zero_hist()

    for r in range(ROUNDS):
        s = wid + r * NWORKERS

        @pl.when(s < NSEG)
        def _():
            base = s * SEG
            pltpu.async_copy(sidx_hbm.at[pl.ds(base, CH)], buf0, sem0)

            def pair(jp, _):
                off = base + jp * (2 * CH)
                pltpu.async_copy(sidx_hbm.at[pl.ds(off + CH, CH)], buf1, sem1)
                wait_dma(buf0, sem0)
                scat_all(buf0)

                @pl.when(jp < NPAIR - 1)
                def _():
                    pltpu.async_copy(
                        sidx_hbm.at[pl.ds(off + 2 * CH, CH)], buf0, sem0)

                wait_dma(buf1, sem1)
                scat_all(buf1)
                return _

            lax.fori_loop(0, NPAIR, pair, None)
            pltpu.sync_copy(hist, out_hbm.at[s])
            zero_hist()


def _sc_hist(sidx_flat):
    mesh = plsc.VectorSubcoreMesh(core_axis_name="c", subcore_axis_name="s")
    fn = functools.partial(
        pl.kernel,
        mesh=mesh,
        out_type=jax.ShapeDtypeStruct((NSEG, HWORDS), jnp.int32),
        scratch_types=[
            pltpu.VMEM((HWORDS,), jnp.int32),
            pltpu.VMEM((CH,), jnp.int32),
            pltpu.VMEM((CH,), jnp.int32),
            pltpu.SemaphoreType.DMA,
            pltpu.SemaphoreType.DMA,
        ],
        compiler_params=pltpu.CompilerParams(
            needs_layout_passes=False, use_tc_tiling_on_sc=True),
    )(_sc_hist_body)
    return fn(sidx_flat)


def _finalize_body(part_ref, out_ref, acc_ref):
    c = pl.program_id(0)
    q = part_ref[0].astype(jnp.float32)
    cnt = jnp.sum(q, axis=0)
    f2 = cnt[K:2 * K].reshape(16, 128)
    n2 = cnt[0:K].reshape(16, 128) + f2

    io = lax.broadcasted_iota(jnp.int32, (128, 128), 0)
    jo = lax.broadcasted_iota(jnp.int32, (128, 128), 1)
    tri_incl = (io >= jo).astype(jnp.float32)
    wn = jnp.dot(n2, tri_incl, preferred_element_type=jnp.float32)
    wf = jnp.dot(f2, tri_incl, preferred_element_type=jnp.float32)
    totn = wn[:, 0:1]
    totf = wf[:, 0:1]
    ia = lax.broadcasted_iota(jnp.int32, (16, 16), 0)
    ja = lax.broadcasted_iota(jnp.int32, (16, 16), 1)
    tri_after = (ja > ia).astype(jnp.float32)
    san = jnp.dot(tri_after, totn, preferred_element_type=jnp.float32)
    saf = jnp.dot(tri_after, totf, preferred_element_type=jnp.float32)
    nn = wn + san
    pp = wf + saf
    g = pp[0, 0]
    jac = 1.0 - (g - pp) / jnp.maximum(g + nn - pp, 1.0)
    loss_c = (jnp.sum(jac) - 0.5 * jac[0, 0]) * (1.0 / float(K))
    present = (g > 0.0).astype(jnp.float32)

    @pl.when(c == 0)
    def _():
        acc_ref[0] = 0.0
        acc_ref[1] = 0.0
        out_ref[0, 0] = 0.0

    acc_ref[0] += loss_c * present
    acc_ref[1] += present

    @pl.when(c == C - 1)
    def _():
        out_ref[0, 0] = acc_ref[0] / jnp.maximum(acc_ref[1], 1.0)


def _finalize(part):
    return pl.pallas_call(
        _finalize_body,
        grid=(C,),
        in_specs=[pl.BlockSpec((1, 128, S), lambda c: (c, 0, 0))],
        out_specs=pl.BlockSpec(memory_space=pltpu.SMEM),
        out_shape=jax.ShapeDtypeStruct((1, 1), jnp.float32),
        scratch_shapes=[pltpu.SMEM((2,), jnp.float32)],
    )(part)


def kernel(logits, labels):
    labels = labels.astype(jnp.int32)
    sidx = _encode(logits, labels)
    part = _sc_hist(sidx.reshape(-1))
    loss = _finalize(part.reshape(C, 128, S))
    return loss[0, 0]


# speedup vs baseline: 2.1176x; 1.0005x over previous
import functools

import jax
import jax.numpy as jnp
from jax import lax
from jax.experimental import pallas as pl
from jax.experimental.pallas import tpu as pltpu
from jax.experimental.pallas import tpu_sc as plsc

B, C, H, W = 4, 19, 512, 512
N = B * H * W
K = 2048
TRASH = 2 * K
S = 4225
HWORDS = 16 * S
SEG_PER_CLASS = 8
SEG = N // SEG_PER_CLASS
NSEG = C * SEG_PER_CLASS
NWORKERS = 32
ROUNDS = (NSEG + NWORKERS - 1) // NWORKERS
CH = 16384
NPAIR = SEG // (2 * CH)
ROWS_BLK = 64


def _encode_body(logits_ref, labels_ref, out_ref):
    x = logits_ref[0]
    lab = labels_ref[0]
    m = jnp.max(x, axis=0)
    ex = jnp.exp(x - m[None])
    inv = 1.0 / jnp.sum(ex, axis=0)
    p = ex * inv[None]
    valid = lab != 0
    cidx = lax.broadcasted_iota(jnp.int32, (C, ROWS_BLK, W), 0)
    fg = (lab[None] == cidx) & valid[None]
    e = jnp.where(fg, 1.0 - p, p)
    bucket = jnp.clip((e * float(K)).astype(jnp.int32), 0, K - 1)
    code = jnp.where(fg, K + bucket, bucket)
    code = jnp.where(valid[None], code, TRASH)
    wio = lax.broadcasted_iota(jnp.int32, (ROWS_BLK, W), 1)
    lane = (wio % 16 + (wio // 16)) % 16
    sidx = code + (lane * S)[None]
    out_ref[...] = sidx.reshape(C, ROWS_BLK * W // 128, 128)


def _encode(logits, labels):
    grid = (B, H // ROWS_BLK)
    return pl.pallas_call(
        _encode_body,
        grid=grid,
        in_specs=[
            pl.BlockSpec((1, C, ROWS_BLK, W), lambda b, r: (b, 0, r, 0)),
            pl.BlockSpec((1, ROWS_BLK, W), lambda b, r: (b, r, 0)),
        ],
        out_specs=pl.BlockSpec(
            (C, ROWS_BLK * W // 128, 128),
            lambda b, r: (0, b * (H // ROWS_BLK) + r, 0)),
        out_shape=jax.ShapeDtypeStruct((C, N // 128, 128), jnp.int32),
    )(logits, labels)


def _sc_hist_body(sidx_hbm, out_hbm, hist, buf0, buf1, sem0, sem1):
    wid = lax.axis_index("s") * 2 + lax.axis_index("c")
    ones = jnp.ones((16,), jnp.int32)
    zeros = jnp.zeros((16,), jnp.int32)

    def zero_hist():
        @plsc.parallel_loop(0, HWORDS // 16, unroll=8)
        def _(i):
            hist[pl.ds(i * 16, 16)] = zeros

    def scat_all(buf):
        @plsc.parallel_loop(0, CH // 16, unroll=8)
        def _(i):
            idx = buf[pl.ds(i * 16, 16)]
            plsc.addupdate_scatter(hist, [idx], ones)

    def wait_dma(buf, sem):
        pltpu.make_async_copy(sidx_hbm.at[pl.ds(0, CH)], buf, sem).wait()

    zero_hist()

    for r in range(ROUNDS):
        s = wid + r * NWORKERS

        @pl.when(s < NSEG)
        def _():
            base = s * SEG
            pltpu.async_copy(sidx_hbm.at[pl.ds(base, CH)], buf0, sem0)

            def pair(jp, _):
                off = base + jp * (2 * CH)
                pltpu.async_copy(sidx_hbm.at[pl.ds(off + CH, CH)], buf1, sem1)
                wait_dma(buf0, sem0)
                scat_all(buf0)

                @pl.when(jp < NPAIR - 1)
                def _():
                    pltpu.async_copy(
                        sidx_hbm.at[pl.ds(off + 2 * CH, CH)], buf0, sem0)

                wait_dma(buf1, sem1)
                scat_all(buf1)
                return _

            lax.fori_loop(0, NPAIR, pair, None)
            pltpu.sync_copy(hist, out_hbm.at[s])
            zero_hist()


def _sc_hist(sidx_flat):
    mesh = plsc.VectorSubcoreMesh(core_axis_name="c", subcore_axis_name="s")
    fn = functools.partial(
        pl.kernel,
        mesh=mesh,
        out_type=jax.ShapeDtypeStruct((NSEG, HWORDS), jnp.int32),
        scratch_types=[
            pltpu.VMEM((HWORDS,), jnp.int32),
            pltpu.VMEM((CH,), jnp.int32),
            pltpu.VMEM((CH,), jnp.int32),
            pltpu.SemaphoreType.DMA,
            pltpu.SemaphoreType.DMA,
        ],
        compiler_params=pltpu.CompilerParams(
            needs_layout_passes=False, use_tc_tiling_on_sc=True),
    )(_sc_hist_body)
    return fn(sidx_flat)


def _finalize_body(part_ref, out_ref, acc_ref):
    c = pl.program_id(0)
    q = part_ref[0].astype(jnp.float32)
    cnt = jnp.sum(q, axis=0)
    f2 = cnt[K:2 * K].reshape(16, 128)
    n2 = cnt[0:K].reshape(16, 128) + f2

    io = lax.broadcasted_iota(jnp.int32, (128, 128), 0)
    jo = lax.broadcasted_iota(jnp.int32, (128, 128), 1)
    tri_incl = (io >= jo).astype(jnp.float32)
    wn = jnp.dot(n2, tri_incl, preferred_element_type=jnp.float32)
    wf = jnp.dot(f2, tri_incl, preferred_element_type=jnp.float32)
    totn = wn[:, 0:1]
    totf = wf[:, 0:1]
    ia = lax.broadcasted_iota(jnp.int32, (16, 16), 0)
    ja = lax.broadcasted_iota(jnp.int32, (16, 16), 1)
    tri_after = (ja > ia).astype(jnp.float32)
    san = jnp.dot(tri_after, totn, preferred_element_type=jnp.float32)
    saf = jnp.dot(tri_after, totf, preferred_element_type=jnp.float32)
    nn = wn + san
    pp = wf + saf
    g = pp[0, 0]
    jac = 1.0 - (g - pp) / jnp.maximum(g + nn - pp, 1.0)
    loss_c = (jnp.sum(jac) - 0.5 * jac[0, 0]) * (1.0 / float(K))
    present = (g > 0.0).astype(jnp.float32)

    @pl.when(c == 0)
    def _():
        acc_ref[0] = 0.0
        acc_ref[1] = 0.0
        out_ref[0, 0] = 0.0

    acc_ref[0] += loss_c * present
    acc_ref[1] += present

    @pl.when(c == C - 1)
    def _():
        out_ref[0, 0] = acc_ref[0] / jnp.maximum(acc_ref[1], 1.0)


def _finalize(part):
    return pl.pallas_call(
        _finalize_body,
        grid=(C,),
        in_specs=[pl.BlockSpec((1, 128, S), lambda c: (c, 0, 0))],
        out_specs=pl.BlockSpec(memory_space=pltpu.SMEM),
        out_shape=jax.ShapeDtypeStruct((1, 1), jnp.float32),
        scratch_shapes=[pltpu.SMEM((2,), jnp.float32)],
    )(part)


def kernel(logits, labels):
    labels = labels.astype(jnp.int32)
    sidx = _encode(logits, labels)
    part = _sc_hist(sidx.reshape(-1))
    loss = _finalize(part.reshape(C, 128, S))
    return loss[0, 0]


# speedup vs baseline: 2.1843x; 1.0315x over previous
import functools

import jax
import jax.numpy as jnp
from jax import lax
from jax.experimental import pallas as pl
from jax.experimental.pallas import tpu as pltpu
from jax.experimental.pallas import tpu_sc as plsc

B, C, H, W = 4, 19, 512, 512
N = B * H * W
K = 2048
TRASH = 2 * K
S = 4225
HWORDS = 16 * S
SEG_PER_CLASS = 8
SEG = N // SEG_PER_CLASS
NSEG = C * SEG_PER_CLASS
NWORKERS = 32
ROUNDS = (NSEG + NWORKERS - 1) // NWORKERS
CH = 16384
NPAIR = SEG // (2 * CH)
ROWS_BLK = 64


def _encode_body(logits_ref, labels_ref, out_ref):
    x = logits_ref[0]
    lab = labels_ref[0]
    m = jnp.max(x, axis=0)
    ex = jnp.exp(x - m[None])
    inv = 1.0 / jnp.sum(ex, axis=0)
    p = ex * inv[None]
    valid = lab != 0
    cidx = lax.broadcasted_iota(jnp.int32, (C, ROWS_BLK, W), 0)
    fg = (lab[None] == cidx) & valid[None]
    e = jnp.where(fg, 1.0 - p, p)
    bucket = jnp.clip((e * float(K)).astype(jnp.int32), 0, K - 1)
    code = jnp.where(fg, K + bucket, bucket)
    code = jnp.where(valid[None], code, TRASH)
    wio = lax.broadcasted_iota(jnp.int32, (ROWS_BLK, W), 1)
    lane = (wio % 16 + (wio // 16)) % 16
    sidx = code + (lane * S)[None]
    out_ref[...] = sidx.reshape(C, ROWS_BLK * W // 128, 128)


def _encode(logits, labels):
    grid = (B, H // ROWS_BLK)
    return pl.pallas_call(
        _encode_body,
        grid=grid,
        in_specs=[
            pl.BlockSpec((1, C, ROWS_BLK, W), lambda b, r: (b, 0, r, 0)),
            pl.BlockSpec((1, ROWS_BLK, W), lambda b, r: (b, r, 0)),
        ],
        out_specs=pl.BlockSpec(
            (C, ROWS_BLK * W // 128, 128),
            lambda b, r: (0, b * (H // ROWS_BLK) + r, 0)),
        out_shape=jax.ShapeDtypeStruct((C, N // 128, 128), jnp.int32),
    )(logits, labels)


def _sc_hist_body(sidx_hbm, out_hbm, hist, buf0, buf1, sem0, sem1):
    wid = lax.axis_index("s") * 2 + lax.axis_index("c")
    ones = jnp.ones((16,), jnp.int32)
    zeros = jnp.zeros((16,), jnp.int32)

    def zero_hist():
        @plsc.parallel_loop(0, HWORDS // 16, unroll=8)
        def _(i):
            hist[pl.ds(i * 16, 16)] = zeros

    def scat_all(buf):
        @plsc.parallel_loop(0, CH // 16, unroll=16)
        def _(i):
            idx = buf[pl.ds(i * 16, 16)]
            plsc.addupdate_scatter(hist, [idx], ones)

    def wait_dma(buf, sem):
        pltpu.make_async_copy(sidx_hbm.at[pl.ds(0, CH)], buf, sem).wait()

    zero_hist()
    first = wid * SEG
    pltpu.async_copy(sidx_hbm.at[pl.ds(first, CH)], buf0, sem0)

    for r in range(ROUNDS):
        s = wid + r * NWORKERS

        @pl.when(s < NSEG)
        def _():
            base = s * SEG

            def pair(jp, _):
                off = base + jp * (2 * CH)
                pltpu.async_copy(sidx_hbm.at[pl.ds(off + CH, CH)], buf1, sem1)
                wait_dma(buf0, sem0)
                scat_all(buf0)

                @pl.when(jp < NPAIR - 1)
                def _():
                    pltpu.async_copy(
                        sidx_hbm.at[pl.ds(off + 2 * CH, CH)], buf0, sem0)

                @pl.when((jp == NPAIR - 1) & (s + NWORKERS < NSEG))
                def _():
                    pltpu.async_copy(
                        sidx_hbm.at[pl.ds((s + NWORKERS) * SEG, CH)],
                        buf0, sem0)

                wait_dma(buf1, sem1)
                scat_all(buf1)
                return _

            lax.fori_loop(0, NPAIR, pair, None)
            pltpu.sync_copy(hist, out_hbm.at[s])
            zero_hist()


def _sc_hist(sidx_flat):
    mesh = plsc.VectorSubcoreMesh(core_axis_name="c", subcore_axis_name="s")
    fn = functools.partial(
        pl.kernel,
        mesh=mesh,
        out_type=jax.ShapeDtypeStruct((NSEG, HWORDS), jnp.int32),
        scratch_types=[
            pltpu.VMEM((HWORDS,), jnp.int32),
            pltpu.VMEM((CH,), jnp.int32),
            pltpu.VMEM((CH,), jnp.int32),
            pltpu.SemaphoreType.DMA,
            pltpu.SemaphoreType.DMA,
        ],
        compiler_params=pltpu.CompilerParams(
            needs_layout_passes=False, use_tc_tiling_on_sc=True),
    )(_sc_hist_body)
    return fn(sidx_flat)


def _finalize_body(part_ref, out_ref, acc_ref):
    c = pl.program_id(0)
    q = part_ref[0].astype(jnp.float32)
    cnt = jnp.sum(q, axis=0)
    f2 = cnt[K:2 * K].reshape(16, 128)
    n2 = cnt[0:K].reshape(16, 128) + f2

    io = lax.broadcasted_iota(jnp.int32, (128, 128), 0)
    jo = lax.broadcasted_iota(jnp.int32, (128, 128), 1)
    tri_incl = (io >= jo).astype(jnp.float32)
    wn = jnp.dot(n2, tri_incl, preferred_element_type=jnp.float32)
    wf = jnp.dot(f2, tri_incl, preferred_element_type=jnp.float32)
    totn = wn[:, 0:1]
    totf = wf[:, 0:1]
    ia = lax.broadcasted_iota(jnp.int32, (16, 16), 0)
    ja = lax.broadcasted_iota(jnp.int32, (16, 16), 1)
    tri_after = (ja > ia).astype(jnp.float32)
    san = jnp.dot(tri_after, totn, preferred_element_type=jnp.float32)
    saf = jnp.dot(tri_after, totf, preferred_element_type=jnp.float32)
    nn = wn + san
    pp = wf + saf
    g = pp[0, 0]
    jac = 1.0 - (g - pp) / jnp.maximum(g + nn - pp, 1.0)
    loss_c = (jnp.sum(jac) - 0.5 * jac[0, 0]) * (1.0 / float(K))
    present = (g > 0.0).astype(jnp.float32)

    @pl.when(c == 0)
    def _():
        acc_ref[0] = 0.0
        acc_ref[1] = 0.0
        out_ref[0, 0] = 0.0

    acc_ref[0] += loss_c * present
    acc_ref[1] += present

    @pl.when(c == C - 1)
    def _():
        out_ref[0, 0] = acc_ref[0] / jnp.maximum(acc_ref[1], 1.0)


def _finalize(part):
    return pl.pallas_call(
        _finalize_body,
        grid=(C,),
        in_specs=[pl.BlockSpec((1, 128, S), lambda c: (c, 0, 0))],
        out_specs=pl.BlockSpec(memory_space=pltpu.SMEM),
        out_shape=jax.ShapeDtypeStruct((1, 1), jnp.float32),
        scratch_shapes=[pltpu.SMEM((2,), jnp.float32)],
    )(part)


def kernel(logits, labels):
    labels = labels.astype(jnp.int32)
    sidx = _encode(logits, labels)
    part = _sc_hist(sidx.reshape(-1))
    loss = _finalize(part.reshape(C, 128, S))
    return loss[0, 0]
